# Initial kernel scaffold; baseline (speedup 1.0000x reference)
import jax
import jax.numpy as jnp
from jax.experimental import pallas as pl


def kernel(feats_0, feats_1, feats_2, feats_3, patch_ids_0, patch_ids_1, patch_ids_2, num_patches):
    raise NotImplementedError("write your pallas kernel here")


# speedup vs baseline: 1.1916x; 1.1916x over previous
import functools

import jax
import jax.numpy as jnp
from jax import lax
from jax.experimental import pallas as pl
from jax.experimental.pallas import tpu as pltpu

try:
    from jax.experimental.pallas import tpu_sc as plsc
    _HAS_SC = True
except ImportError:
    plsc = None
    _HAS_SC = False


def _dots_body(feat_pad_ref, feat_ctr_ref, dots_ref):
    offs = [(di, dj) for di in range(7) for dj in range(7)]
    c_io = lax.broadcasted_iota(jnp.int32, (256, 256), 0)
    col_io = lax.broadcasted_iota(jnp.int32, (256, 256), 1)
    c49 = lax.broadcasted_iota(jnp.int32, (256, 49), 0)
    o49 = lax.broadcasted_iota(jnp.int32, (256, 49), 1)
    perms = [(c_io == (49 * col_io + o) % 256).astype(jnp.float32)
             for o in range(49)]
    gmats = [((49 * c49 + o) // 256 == o49).astype(jnp.float32)
             for o in range(49)]

    for b in range(2):
        fp = feat_pad_ref[b]
        q = feat_ctr_ref[b]
        q_bf = q.astype(jnp.bfloat16).astype(jnp.float32)
        dots_all = jnp.zeros((1024, 49), jnp.float32)
        for o, (di, dj) in enumerate(offs):
            sh = fp[di:di + 32, dj:dj + 32, :].reshape(1024, 256)
            sh_bf = sh.astype(jnp.bfloat16).astype(jnp.float32)
            qperm = jnp.dot(q_bf, perms[o],
                            precision=lax.Precision.HIGHEST,
                            preferred_element_type=jnp.float32)
            dots_all = dots_all + jnp.dot(
                sh_bf * qperm, gmats[o],
                precision=lax.Precision.HIGHEST,
                preferred_element_type=jnp.float32)
        dots_ref[b] = dots_all


def _tc_body(feat_flat_ref, dots_ref, ent_ref, raw_ref, out_ref):
    raw = raw_ref[...]
    norm = jnp.sqrt(jnp.sum(raw * raw, axis=-1, keepdims=True))
    out_ref[0:3, :, :] = raw / (norm + 1e-7)

    flat_i = lax.broadcasted_iota(jnp.int32, (1024, 1), 0)
    k_iota = lax.broadcasted_iota(jnp.int32, (64, 1), 0)
    lane1024 = lax.broadcasted_iota(jnp.int32, (64, 1024), 1)
    lane1600 = lax.broadcasted_iota(jnp.int32, (64, 1600), 1)
    offs = [(di, dj) for di in range(7) for dj in range(7)]
    eye64 = (lax.broadcasted_iota(jnp.int32, (64, 64), 0)
             == lax.broadcasted_iota(jnp.int32, (64, 64), 1)).astype(jnp.float32)
    c_io = lax.broadcasted_iota(jnp.int32, (256, 256), 0)
    col_io = lax.broadcasted_iota(jnp.int32, (256, 256), 1)
    scat = (c_io == 4 * (col_io % 64) + col_io // 64).astype(jnp.float32)

    for b in range(2):
        dots_all = dots_ref[b]
        ent = ent_ref[b]

        def select_body(k, carry):
            ent_c, sel_ids = carry
            mval = jnp.min(ent_c)
            sel = jnp.min(jnp.where(ent_c == mval, flat_i, jnp.int32(1 << 30)))
            sel_ids = jnp.where(k_iota == k, sel, sel_ids)
            ent_c = jnp.where(flat_i == sel, jnp.inf, ent_c)
            return ent_c, sel_ids

        _, sel_ids = lax.fori_loop(
            0, 64, select_body,
            (ent, jnp.zeros((64, 1), jnp.int32)), unroll=False)

        hot_pix = (sel_ids == lane1024).astype(jnp.float32)
        dsel = jnp.dot(hot_pix, dots_all,
                       precision=lax.Precision.HIGHEST,
                       preferred_element_type=jnp.float32)
        dmax = jnp.max(dsel, axis=1, keepdims=True)
        esel = jnp.exp(dsel - dmax)
        attn = esel / jnp.sum(esel, axis=1, keepdims=True)
        i_ids = sel_ids // 32
        j_ids = sel_ids % 32
        attn_t = lax.dot_general(attn, eye64, (((0,), (0,)), ((), ())),
                                 precision=lax.Precision.HIGHEST,
                                 preferred_element_type=jnp.float32)

        fs = jnp.dot(feat_flat_ref[b], scat,
                     precision=lax.Precision.HIGHEST,
                     preferred_element_type=jnp.float32)
        fs = fs.astype(jnp.bfloat16).astype(jnp.float32)
        base = i_ids * 40 + j_ids
        accs = [jnp.zeros((64, 64), jnp.float32) for _ in range(4)]
        for o, (di, dj) in enumerate(offs):
            hot = (base + (di * 40 + dj) == lane1600).astype(jnp.float32)
            zc = jnp.dot(hot, fs, precision=lax.Precision.HIGHEST,
                         preferred_element_type=jnp.float32)
            for t in range(4):
                n = 49 * t + o
                accs[n % 4] = accs[n % 4] + (zc[:, 64 * t:64 * (t + 1)]
                                             * attn_t[n // 4:n // 4 + 1, :])

        ssq = jnp.zeros((1, 64), jnp.float32)
        for a in accs:
            ssq = ssq + jnp.sum(a * a, axis=0, keepdims=True)
        inv = 1.0 / (jnp.sqrt(ssq) + 1e-7)
        for h in range(4):
            blk = lax.dot_general(accs[h] * inv, eye64,
                                  (((0,), (0,)), ((), ())),
                                  precision=lax.Precision.HIGHEST,
                                  preferred_element_type=jnp.float32)
            out_ref[3, b * 64:(b + 1) * 64, 64 * h:64 * (h + 1)] = blk


def _run_dots(feat_pad, feat_ctr, interpret=False):
    return pl.pallas_call(
        _dots_body,
        out_shape=jax.ShapeDtypeStruct((2, 1024, 49), jnp.float32),
        interpret=interpret,
    )(feat_pad, feat_ctr)


def _run_tc(feat_flat, dots, ent, raw012, interpret=False):
    return pl.pallas_call(
        _tc_body,
        out_shape=jax.ShapeDtypeStruct((4, 128, 256), jnp.float32),
        interpret=interpret,
    )(feat_flat, dots, ent, raw012)


_N_PER_FEAT = 2 * 64 * 256


def _sc_gather(t0, t1, t2, idx0, idx1, idx2):
    info = plsc.get_sparse_core_info()
    nw = info.num_cores * info.num_subcores
    per_w = _N_PER_FEAT // nw
    n_chunk = 128
    chunks = per_w // n_chunk

    mesh = plsc.VectorSubcoreMesh(core_axis_name="c", subcore_axis_name="s")

    @functools.partial(
        pl.kernel, mesh=mesh,
        out_type=jax.ShapeDtypeStruct((3 * _N_PER_FEAT,), jnp.float32),
        scratch_types=[
            pltpu.VMEM((per_w,), jnp.int32),
            pltpu.VMEM((per_w,), jnp.float32),
            pltpu.SemaphoreType.DMA,
        ],
    )
    def k(t0_hbm, t1_hbm, t2_hbm, i0_hbm, i1_hbm, i2_hbm, out_hbm,
          idx_v, val_v, sem):
        wid = lax.axis_index("s") * info.num_cores + lax.axis_index("c")
        base = wid * per_w
        for f, (t_hbm, i_hbm) in enumerate(
                ((t0_hbm, i0_hbm), (t1_hbm, i1_hbm), (t2_hbm, i2_hbm))):
            pltpu.sync_copy(i_hbm.at[pl.ds(base, per_w)], idx_v)

            def chunk_body(c, _, t_hbm=t_hbm):
                off = c * n_chunk
                pltpu.async_copy(
                    t_hbm.at[idx_v.at[pl.ds(off, n_chunk)]],
                    val_v.at[pl.ds(off, n_chunk)], sem).wait()
                return 0

            lax.fori_loop(0, chunks, chunk_body, 0, unroll=False)
            pltpu.sync_copy(val_v,
                            out_hbm.at[pl.ds(f * _N_PER_FEAT + base, per_w)])

    return k(t0, t1, t2, idx0, idx1, idx2)


def _flat_indices(pid, C, HW):
    pid = pid.astype(jnp.int32)
    b = jnp.arange(2, dtype=jnp.int32)[:, None, None]
    c = jnp.arange(C, dtype=jnp.int32)[None, None, :]
    return (b * (C * HW) + c * HW + pid[None, :, None]).reshape(-1)


def kernel(feats_0, feats_1, feats_2, feats_3, patch_ids_0, patch_ids_1,
           patch_ids_2, num_patches):

    f3 = jnp.transpose(feats_3, (0, 2, 3, 1))
    feat_pad = jnp.pad(f3, ((0, 0), (3, 5), (3, 5), (0, 0)))
    feat_flat = feat_pad.reshape(2, 1600, 256)
    feat_ctr = f3.reshape(2, 1024, 256)

    idx0 = _flat_indices(patch_ids_0, 256, 128 * 128)
    idx1 = _flat_indices(patch_ids_1, 256, 64 * 64)
    idx2 = _flat_indices(patch_ids_2, 256, 64 * 64)

    raw = _sc_gather(feats_0.reshape(-1), feats_1.reshape(-1),
                     feats_2.reshape(-1), idx0, idx1, idx2)
    raw012 = raw.reshape(3, 2 * 64, 256)

    dots = _run_dots(feat_pad, feat_ctr)
    dots_local = dots.reshape(2 * 1024, 49, 1)
    attn_local = jax.nn.softmax(dots_local, axis=1).reshape(2, 1024, 49)
    prob = -jnp.log(attn_local)
    prob = jnp.where(jnp.isinf(prob), jnp.zeros_like(prob), prob)
    entropy = jnp.sum(attn_local * prob, axis=2) + num_patches * 0

    return _run_tc(feat_flat, dots, entropy.reshape(2, 1024, 1), raw012)
